# baseline (speedup 1.0000x reference)
import jax
import jax.numpy as jnp
from jax.experimental import pallas as pl
from jax.experimental.pallas import tpu as pltpu

_T = 32
_BL = 3
_P = 8
_MB = 4
_BB = 16
_NBUF = 4


def _burst_body(x_hbm, out_hbm, xv, buf, in_sem, out_sem):
    cp_in = pltpu.make_async_copy(x_hbm, xv, in_sem)
    cp_in.start()

    f = xv.shape[-1]
    nb = x_hbm.shape[0] // _BB
    t = jax.lax.broadcasted_iota(jnp.int32, (_T, f), 0)
    thr = ((t // _P) + 1).astype(jnp.float32)
    within = (t % _P) < _BL

    cp_in.wait()

    def compute_block(i, slot):
        def bbody(b, carry):
            v = xv[pl.ds(i * _BB + b, 1), :] * jnp.float32(_MB)
            act = jnp.broadcast_to(v, (_T, f)) >= thr
            buf[slot, b] = (within & act).astype(jnp.float32)
            return carry
        jax.lax.fori_loop(0, _BB, bbody, 0, unroll=True)

    def start_out(i, slot):
        pltpu.make_async_copy(
            buf.at[slot], out_hbm.at[pl.ds(i * _BB, _BB)], out_sem.at[slot]
        ).start()

    def wait_out(i, slot):
        pltpu.make_async_copy(
            buf.at[slot], out_hbm.at[pl.ds(i * _BB, _BB)], out_sem.at[slot]
        ).wait()

    for j in range(_NBUF):
        compute_block(j, j)
        start_out(j, j)

    def loop_body(i2, carry):
        for j in range(_NBUF):
            i = i2 * _NBUF + j
            wait_out(i - _NBUF, j)
            compute_block(i, j)
            start_out(i, j)
        return carry

    jax.lax.fori_loop(1, nb // _NBUF, loop_body, 0)

    for j in range(_NBUF):
        wait_out(nb - _NBUF + j, j)


def kernel(x):
    bsz, f = x.shape
    return pl.pallas_call(
        _burst_body,
        in_specs=[pl.BlockSpec(memory_space=pl.ANY)],
        out_specs=pl.BlockSpec(memory_space=pl.ANY),
        out_shape=jax.ShapeDtypeStruct((bsz, _T, f), jnp.float32),
        scratch_shapes=[
            pltpu.VMEM((bsz, f), jnp.float32),
            pltpu.VMEM((_NBUF, _BB, _T, f), jnp.float32),
            pltpu.SemaphoreType.DMA,
            pltpu.SemaphoreType.DMA((_NBUF,)),
        ],
    )(x)


# speedup vs baseline: 1.0165x; 1.0165x over previous
import jax
import jax.numpy as jnp
from jax.experimental import pallas as pl
from jax.experimental.pallas import tpu as pltpu

_T = 32
_BL = 3
_P = 8
_MB = 4
_BB = 8
_NBUF = 8


def _burst_body(x_hbm, out_hbm, xv, buf, in_sem, out_sem):
    cp_in = pltpu.make_async_copy(x_hbm, xv, in_sem)
    cp_in.start()

    f = xv.shape[-1]
    nb = x_hbm.shape[0] // _BB
    t = jax.lax.broadcasted_iota(jnp.int32, (_T, f), 0)
    thr = ((t // _P) + 1).astype(jnp.float32)
    within = (t % _P) < _BL

    cp_in.wait()

    def compute_block(i, slot):
        def bbody(b, carry):
            v = xv[pl.ds(i * _BB + b, 1), :] * jnp.float32(_MB)
            act = jnp.broadcast_to(v, (_T, f)) >= thr
            buf[slot, b] = (within & act).astype(jnp.float32)
            return carry
        jax.lax.fori_loop(0, _BB, bbody, 0, unroll=True)

    def start_out(i, slot):
        pltpu.make_async_copy(
            buf.at[slot], out_hbm.at[pl.ds(i * _BB, _BB)], out_sem.at[slot]
        ).start()

    def wait_out(i, slot):
        pltpu.make_async_copy(
            buf.at[slot], out_hbm.at[pl.ds(i * _BB, _BB)], out_sem.at[slot]
        ).wait()

    for j in range(_NBUF):
        compute_block(j, j)
        start_out(j, j)

    def loop_body(i2, carry):
        for j in range(_NBUF):
            i = i2 * _NBUF + j
            wait_out(i - _NBUF, j)
            compute_block(i, j)
            start_out(i, j)
        return carry

    jax.lax.fori_loop(1, nb // _NBUF, loop_body, 0)

    for j in range(_NBUF):
        wait_out(nb - _NBUF + j, j)


def kernel(x):
    bsz, f = x.shape
    return pl.pallas_call(
        _burst_body,
        in_specs=[pl.BlockSpec(memory_space=pl.ANY)],
        out_specs=pl.BlockSpec(memory_space=pl.ANY),
        out_shape=jax.ShapeDtypeStruct((bsz, _T, f), jnp.float32),
        scratch_shapes=[
            pltpu.VMEM((bsz, f), jnp.float32),
            pltpu.VMEM((_NBUF, _BB, _T, f), jnp.float32),
            pltpu.SemaphoreType.DMA,
            pltpu.SemaphoreType.DMA((_NBUF,)),
        ],
    )(x)


# speedup vs baseline: 1.0547x; 1.0376x over previous
import jax
import jax.numpy as jnp
from jax.experimental import pallas as pl

_T = 32
_BL = 3
_P = 8
_MB = 4
_BB = 16


def _burst_body(x_ref, out_ref):
    v = x_ref[...] * jnp.float32(_MB)
    f = v.shape[-1]
    t = jax.lax.broadcasted_iota(jnp.int32, (_T, f), 0)
    thr = ((t // _P) + 1).astype(jnp.float32)
    within = (t % _P) < _BL
    for b in range(v.shape[0]):
        act = jnp.broadcast_to(v[b:b + 1, :], (_T, f)) >= thr
        out_ref[b, :, :] = (within & act).astype(jnp.float32)


def kernel(x):
    bsz, f = x.shape
    grid = (bsz // _BB,)
    return pl.pallas_call(
        _burst_body,
        grid=grid,
        in_specs=[pl.BlockSpec((_BB, f), lambda i: (i, 0))],
        out_specs=pl.BlockSpec((_BB, _T, f), lambda i: (i, 0, 0)),
        out_shape=jax.ShapeDtypeStruct((bsz, _T, f), jnp.float32),
    )(x)
